# Initial kernel scaffold; baseline (speedup 1.0000x reference)
import jax
import jax.numpy as jnp
from jax.experimental import pallas as pl


def kernel(inputs, embed, bias):
    raise NotImplementedError("write your pallas kernel here")


# speedup vs baseline: 16.3687x; 16.3687x over previous
import functools

import jax
import jax.numpy as jnp
from jax import lax
from jax.experimental import pallas as pl
from jax.experimental.pallas import tpu as pltpu
from jax.experimental.pallas import tpu_sc as plsc

B = 16384
L = 200
D = 32

NC = 2
NS = 16
NW = NC * NS
BPW = B // NW
CB = 8
NCH = BPW // CB
ROWS = CB * L
HALF = D // 2


def _bow_kernel(idx_hbm, embed_hbm, bias_hbm, out_hbm,
                idx_v, rows_v, acc_v, bias_v, sem0, sem1):
    wid = lax.axis_index("s") * NC + lax.axis_index("c")
    base_row = wid * BPW
    sems = (sem0, sem1)

    pltpu.sync_copy(bias_hbm, bias_v)
    bias_lo = bias_v[0:HALF]
    bias_hi = bias_v[HALF:D]

    def load_and_fire(ch, slot):
        flat_base = (base_row + ch * CB) * L
        pltpu.sync_copy(idx_hbm.at[pl.ds(flat_base, ROWS)],
                        idx_v.at[pl.ds(slot * ROWS, ROWS)])
        return pltpu.async_copy(embed_hbm.at[idx_v.at[pl.ds(slot * ROWS, ROWS)]],
                                rows_v.at[slot], sems[slot])

    def accumulate(ch, slot):
        for r in range(CB):
            rb = r * L

            def body(j, carry):
                a0, a1 = carry
                a0 = a0 + rows_v[slot, rb + j, 0:HALF]
                a1 = a1 + rows_v[slot, rb + j, HALF:D]
                return a0, a1

            a0, a1 = pl.loop(0, L, init_carry=(bias_lo, bias_hi),
                             unroll=8)(body)
            acc_v[r, 0:HALF] = a0
            acc_v[r, HALF:D] = a1
        pltpu.sync_copy(acc_v, out_hbm.at[pl.ds(base_row + ch * CB, CB), :])

    load_and_fire(0, 0)

    @pl.loop(0, NCH, step=2)
    def _(c):
        for s in range(2):
            ch = c + s

            @pl.when(ch + 1 < NCH)
            def _():
                load_and_fire(ch + 1, (s + 1) % 2)

            pltpu.make_async_copy(
                embed_hbm.at[idx_v.at[pl.ds(s * ROWS, ROWS)]],
                rows_v.at[s], sems[s]).wait()
            accumulate(ch, s)


@jax.jit
def kernel(inputs, embed, bias):
    idx_flat = inputs.reshape(B * L).astype(jnp.int32)
    mesh = plsc.VectorSubcoreMesh(core_axis_name="c", subcore_axis_name="s")
    run = pl.kernel(
        _bow_kernel,
        out_type=jax.ShapeDtypeStruct((B, D), jnp.float32),
        mesh=mesh,
        scratch_types=[
            pltpu.VMEM((2 * ROWS,), jnp.int32),
            pltpu.VMEM((2, ROWS, D), jnp.float32),
            pltpu.VMEM((CB, D), jnp.float32),
            pltpu.VMEM((D,), jnp.float32),
            pltpu.SemaphoreType.DMA,
            pltpu.SemaphoreType.DMA,
        ],
        compiler_params=pltpu.CompilerParams(use_tc_tiling_on_sc=False),
    )
    return run(idx_flat, embed, bias)


# speedup vs baseline: 16.3702x; 1.0001x over previous
import functools

import jax
import jax.numpy as jnp
from jax import lax
from jax.experimental import pallas as pl
from jax.experimental.pallas import tpu as pltpu
from jax.experimental.pallas import tpu_sc as plsc

B = 16384
L = 200
D = 32

NC = 2
NS = 16
NW = NC * NS
BPW = B // NW
CB = 8
NCH = BPW // CB
ROWS = CB * L
HALF = D // 2


def _bow_kernel(idx_hbm, embed_hbm, bias_hbm, out_hbm,
                idx_v, rows_v, acc_v, bias_v, sem0, sem1, isem0, isem1):
    wid = lax.axis_index("s") * NC + lax.axis_index("c")
    base_row = wid * BPW
    sems = (sem0, sem1)
    isems = (isem0, isem1)

    pltpu.sync_copy(bias_hbm, bias_v)
    bias_lo = bias_v[0:HALF]
    bias_hi = bias_v[HALF:D]

    def load_and_fire(ch, slot):
        row0 = base_row + ch * CB
        for r in range(CB):
            pltpu.async_copy(idx_hbm.at[row0 + r, :],
                             idx_v.at[pl.ds(slot * ROWS + r * L, L)],
                             isems[slot])
        for r in range(CB):
            pltpu.make_async_copy(idx_hbm.at[row0 + r, :],
                                  idx_v.at[pl.ds(slot * ROWS + r * L, L)],
                                  isems[slot]).wait()
        return pltpu.async_copy(embed_hbm.at[idx_v.at[pl.ds(slot * ROWS, ROWS)]],
                                rows_v.at[slot], sems[slot])

    def accumulate(ch, slot):
        for r in range(CB):
            rb = r * L

            def body(j, carry):
                a0, a1 = carry
                a0 = a0 + rows_v[slot, rb + j, 0:HALF]
                a1 = a1 + rows_v[slot, rb + j, HALF:D]
                return a0, a1

            a0, a1 = pl.loop(0, L, init_carry=(bias_lo, bias_hi),
                             unroll=8)(body)
            acc_v[r, 0:HALF] = a0
            acc_v[r, HALF:D] = a1
        pltpu.sync_copy(acc_v, out_hbm.at[pl.ds(base_row + ch * CB, CB), :])

    load_and_fire(0, 0)

    @pl.loop(0, NCH, step=2)
    def _(c):
        for s in range(2):
            ch = c + s

            @pl.when(ch + 1 < NCH)
            def _():
                load_and_fire(ch + 1, (s + 1) % 2)

            pltpu.make_async_copy(
                embed_hbm.at[idx_v.at[pl.ds(s * ROWS, ROWS)]],
                rows_v.at[s], sems[s]).wait()
            accumulate(ch, s)


@jax.jit
def kernel(inputs, embed, bias):
    idx2d = inputs.astype(jnp.int32)
    mesh = plsc.VectorSubcoreMesh(core_axis_name="c", subcore_axis_name="s")
    run = pl.kernel(
        _bow_kernel,
        out_type=jax.ShapeDtypeStruct((B, D), jnp.float32),
        mesh=mesh,
        scratch_types=[
            pltpu.VMEM((2 * ROWS,), jnp.int32),
            pltpu.VMEM((2, ROWS, D), jnp.float32),
            pltpu.VMEM((CB, D), jnp.float32),
            pltpu.VMEM((D,), jnp.float32),
            pltpu.SemaphoreType.DMA,
            pltpu.SemaphoreType.DMA,
            pltpu.SemaphoreType.DMA,
            pltpu.SemaphoreType.DMA,
        ],
        compiler_params=pltpu.CompilerParams(use_tc_tiling_on_sc=False),
    )
    return run(idx2d, embed, bias)


# speedup vs baseline: 16.6497x; 1.0171x over previous
import functools

import jax
import jax.numpy as jnp
from jax import lax
from jax.experimental import pallas as pl
from jax.experimental.pallas import tpu as pltpu
from jax.experimental.pallas import tpu_sc as plsc

B = 16384
L = 200
D = 32
VOCAB = 1000000

NC = 2
NS = 16
NW = NC * NS
BPW = B // NW
CB = 8
NCH = BPW // CB
ROWS = CB * L
HALF = D // 2


def _bow_kernel(idx_hbm, embed_hbm, bias_hbm, out_hbm,
                idx_v, rows_v, acc_v, bias_v, sem0, sem1, isem0, isem1):
    wid = lax.axis_index("s") * NC + lax.axis_index("c")
    base_row = wid * BPW
    sems = (sem0, sem1)
    isems = (isem0, isem1)

    pltpu.sync_copy(bias_hbm, bias_v)
    lane = lax.iota(jnp.int32, HALF)
    bias_even = plsc.load_gather(bias_v, [lane * 2])
    bias_odd = plsc.load_gather(bias_v, [lane * 2 + 1])

    def load_and_fire(ch, slot):
        row0 = base_row + ch * CB
        for r in range(CB):
            pltpu.async_copy(idx_hbm.at[row0 + r, :],
                             idx_v.at[pl.ds(slot * ROWS + r * L, L)],
                             isems[slot])
        for r in range(CB):
            pltpu.make_async_copy(idx_hbm.at[row0 + r, :],
                                  idx_v.at[pl.ds(slot * ROWS + r * L, L)],
                                  isems[slot]).wait()
        return pltpu.async_copy(embed_hbm.at[idx_v.at[pl.ds(slot * ROWS, ROWS)]],
                                rows_v.at[slot], sems[slot])

    def accumulate(ch, slot):
        for r in range(CB):
            rb = r * L

            def body(j, carry):
                a_even, a_odd = carry
                w = plsc.bitcast(rows_v[slot, rb + j, :], jnp.int32)
                a_even = a_even + plsc.bitcast(w << 16, jnp.float32)
                a_odd = a_odd + plsc.bitcast(w, jnp.float32)
                return a_even, a_odd

            a_even, a_odd = pl.loop(0, L, init_carry=(bias_even, bias_odd),
                                    unroll=8)(body)
            ridx = jnp.full((HALF,), r, jnp.int32)
            plsc.store_scatter(acc_v, [ridx, lane * 2], a_even)
            plsc.store_scatter(acc_v, [ridx, lane * 2 + 1], a_odd)
        pltpu.sync_copy(acc_v, out_hbm.at[pl.ds(base_row + ch * CB, CB), :])

    load_and_fire(0, 0)

    @pl.loop(0, NCH, step=2)
    def _(c):
        for s in range(2):
            ch = c + s

            @pl.when(ch + 1 < NCH)
            def _():
                load_and_fire(ch + 1, (s + 1) % 2)

            pltpu.make_async_copy(
                embed_hbm.at[idx_v.at[pl.ds(s * ROWS, ROWS)]],
                rows_v.at[s], sems[s]).wait()
            accumulate(ch, s)


_TC_CC = 16384
_TC_QQ = _TC_CC // 4


_TC_NCHUNK = 4
_TC_SC = _TC_CC // _TC_NCHUNK
_TC_SQ = _TC_SC // 4


def _pack_body(in_ref, out_ref, *ys):
    for c in range(_TC_NCHUNK):
        ys[c][...] = in_ref[:, c * _TC_SC:(c + 1) * _TC_SC].T
    for c in range(_TC_NCHUNK):
        for j in range(4):
            out_ref[c * _TC_SQ:(c + 1) * _TC_SQ, D * j:D * (j + 1)] = (
                ys[c][pl.Slice(j, _TC_SQ, 4), :].astype(jnp.bfloat16))


def _pack_table(embed):
    return pl.pallas_call(
        _pack_body,
        grid=((VOCAB + _TC_CC - 1) // _TC_CC,),
        in_specs=[pl.BlockSpec((D, _TC_CC), lambda c: (0, c))],
        out_specs=pl.BlockSpec((_TC_QQ, 128), lambda c: (c, 0)),
        out_shape=jax.ShapeDtypeStruct((VOCAB // 4, 128), jnp.bfloat16),
        scratch_shapes=[pltpu.VMEM((_TC_SC, D), jnp.float32)
                        for _ in range(_TC_NCHUNK)],
    )(embed.T)


@jax.jit
def kernel(inputs, embed, bias):
    idx2d = inputs.astype(jnp.int32)
    tbl = _pack_table(embed).reshape(VOCAB, D)
    mesh = plsc.VectorSubcoreMesh(core_axis_name="c", subcore_axis_name="s")
    run = pl.kernel(
        _bow_kernel,
        out_type=jax.ShapeDtypeStruct((B, D), jnp.float32),
        mesh=mesh,
        scratch_types=[
            pltpu.VMEM((2 * ROWS,), jnp.int32),
            pltpu.VMEM((2, ROWS, D), jnp.bfloat16),
            pltpu.VMEM((CB, D), jnp.float32),
            pltpu.VMEM((D,), jnp.float32),
            pltpu.SemaphoreType.DMA,
            pltpu.SemaphoreType.DMA,
            pltpu.SemaphoreType.DMA,
            pltpu.SemaphoreType.DMA,
        ],
        compiler_params=pltpu.CompilerParams(use_tc_tiling_on_sc=False,
                                             needs_layout_passes=False),
    )
    return run(idx2d, tbl, bias)


# speedup vs baseline: 22.6311x; 1.3593x over previous
import functools

import jax
import jax.numpy as jnp
from jax import lax
from jax.experimental import pallas as pl
from jax.experimental.pallas import tpu as pltpu
from jax.experimental.pallas import tpu_sc as plsc

B = 16384
L = 200
D = 32
VOCAB = 1000000

NC = 2
NS = 16
NW = NC * NS
BPW = B // NW
CB = 8
NCH = BPW // CB
ROWS = CB * L
HALF = D // 2


def _bow_kernel(idx_hbm, embed_hbm, bias_hbm, out_hbm,
                idx_v, rows_v, acc_v, bias_v, sem0, sem1, isem0, isem1,
                osem0, osem1):
    wid = lax.axis_index("s") * NC + lax.axis_index("c")
    base_row = wid * BPW
    sems = (sem0, sem1)
    isems = (isem0, isem1)
    osems = (osem0, osem1)

    pltpu.sync_copy(bias_hbm, bias_v)
    bias_lo = bias_v[0:HALF]
    bias_hi = bias_v[HALF:D]

    def load_and_fire(ch, slot):
        row0 = base_row + ch * CB
        for r in range(CB):
            pltpu.async_copy(idx_hbm.at[row0 + r, :],
                             idx_v.at[pl.ds(slot * ROWS + r * L, L)],
                             isems[slot])
        for r in range(CB):
            pltpu.make_async_copy(idx_hbm.at[row0 + r, :],
                                  idx_v.at[pl.ds(slot * ROWS + r * L, L)],
                                  isems[slot]).wait()
        return pltpu.async_copy(embed_hbm.at[idx_v.at[pl.ds(slot * ROWS, ROWS)]],
                                rows_v.at[slot], sems[slot])

    def accumulate(ch, slot):
        @pl.when(ch >= 2)
        def _():
            pltpu.make_async_copy(
                acc_v.at[slot],
                out_hbm.at[pl.ds(base_row + (ch - 2) * CB, CB), :],
                osems[slot]).wait()

        for r in range(CB):
            rb = r * L

            def body(j, carry):
                a0, a1 = carry
                a0 = a0 + rows_v[slot, rb + j, 0:HALF]
                a1 = a1 + rows_v[slot, rb + j, HALF:D]
                return a0, a1

            a0, a1 = pl.loop(0, L, init_carry=(bias_lo, bias_hi),
                             unroll=25)(body)
            acc_v[slot, r, 0:HALF] = a0
            acc_v[slot, r, HALF:D] = a1
        pltpu.async_copy(acc_v.at[slot],
                         out_hbm.at[pl.ds(base_row + ch * CB, CB), :],
                         osems[slot])

    load_and_fire(0, 0)

    @pl.loop(0, NCH, step=2)
    def _(c):
        for s in range(2):
            ch = c + s

            @pl.when(ch + 1 < NCH)
            def _():
                load_and_fire(ch + 1, (s + 1) % 2)

            pltpu.make_async_copy(
                embed_hbm.at[idx_v.at[pl.ds(s * ROWS, ROWS)]],
                rows_v.at[s], sems[s]).wait()
            accumulate(ch, s)

    for s in range(2):
        pltpu.make_async_copy(
            acc_v.at[s],
            out_hbm.at[pl.ds(base_row + (NCH - 2 + s) * CB, CB), :],
            osems[s]).wait()


_TC_CC = 16384
_TC_QQ = _TC_CC // 4


_TC_NCHUNK = 4
_TC_SC = _TC_CC // _TC_NCHUNK
_TC_SQ = _TC_SC // 4


def _pack_body(in_ref, out_ref, *ys):
    for c in range(_TC_NCHUNK):
        ys[c][...] = in_ref[:, c * _TC_SC:(c + 1) * _TC_SC].T
    for c in range(_TC_NCHUNK):
        for j in range(4):
            out_ref[c * _TC_SQ:(c + 1) * _TC_SQ, D * j:D * (j + 1)] = (
                ys[c][pl.Slice(j, _TC_SQ, 4), :])


def _pack_table(embed):
    return pl.pallas_call(
        _pack_body,
        grid=((VOCAB + _TC_CC - 1) // _TC_CC,),
        in_specs=[pl.BlockSpec((D, _TC_CC), lambda c: (0, c))],
        out_specs=pl.BlockSpec((_TC_QQ, 128), lambda c: (c, 0)),
        out_shape=jax.ShapeDtypeStruct((VOCAB // 4, 128), jnp.float32),
        scratch_shapes=[pltpu.VMEM((_TC_SC, D), jnp.float32)
                        for _ in range(_TC_NCHUNK)],
    )(embed.T)


@jax.jit
def kernel(inputs, embed, bias):
    idx2d = inputs.astype(jnp.int32)
    tbl = _pack_table(embed).reshape(VOCAB, D)
    mesh = plsc.VectorSubcoreMesh(core_axis_name="c", subcore_axis_name="s")
    run = pl.kernel(
        _bow_kernel,
        out_type=jax.ShapeDtypeStruct((B, D), jnp.float32),
        mesh=mesh,
        scratch_types=[
            pltpu.VMEM((2 * ROWS,), jnp.int32),
            pltpu.VMEM((2, ROWS, D), jnp.float32),
            pltpu.VMEM((2, CB, D), jnp.float32),
            pltpu.VMEM((D,), jnp.float32),
            pltpu.SemaphoreType.DMA,
            pltpu.SemaphoreType.DMA,
            pltpu.SemaphoreType.DMA,
            pltpu.SemaphoreType.DMA,
            pltpu.SemaphoreType.DMA,
            pltpu.SemaphoreType.DMA,
        ],
        compiler_params=pltpu.CompilerParams(use_tc_tiling_on_sc=False),
    )
    return run(idx2d, tbl, bias)


# speedup vs baseline: 23.1858x; 1.0245x over previous
import functools

import jax
import jax.numpy as jnp
from jax import lax
from jax.experimental import pallas as pl
from jax.experimental.pallas import tpu as pltpu
from jax.experimental.pallas import tpu_sc as plsc

B = 16384
L = 200
D = 32
VOCAB = 1000000

NC = 2
NS = 16
NW = NC * NS
BPW = B // NW
CB = 8
NCH = BPW // CB
ROWS = CB * L
HALF = D // 2


def _bow_kernel(idx_hbm, embed_hbm, bias_hbm, out_hbm,
                idx_v, rows_v, acc_v, bias_v, sem0, sem1, isem0, isem1,
                osem0, osem1):
    wid = lax.axis_index("s") * NC + lax.axis_index("c")
    base_row = wid * BPW
    sems = (sem0, sem1)
    isems = (isem0, isem1)
    osems = (osem0, osem1)

    pltpu.sync_copy(bias_hbm, bias_v)
    bias_lo = bias_v[0:HALF]
    bias_hi = bias_v[HALF:D]

    def load_and_fire(ch, slot):
        row0 = base_row + ch * CB
        for r in range(CB):
            pltpu.async_copy(idx_hbm.at[row0 + r, :],
                             idx_v.at[pl.ds(slot * ROWS + r * L, L)],
                             isems[slot])
        for r in range(CB):
            pltpu.make_async_copy(idx_hbm.at[row0 + r, :],
                                  idx_v.at[pl.ds(slot * ROWS + r * L, L)],
                                  isems[slot]).wait()
        return pltpu.async_copy(embed_hbm.at[idx_v.at[pl.ds(slot * ROWS, ROWS)]],
                                rows_v.at[slot], sems[slot])

    def accumulate(ch, slot):
        @pl.when(ch >= 2)
        def _():
            pltpu.make_async_copy(
                acc_v.at[slot],
                out_hbm.at[pl.ds(base_row + (ch - 2) * CB, CB), :],
                osems[slot]).wait()

        for r in range(CB):
            rb = r * L

            def body(j, carry):
                a0, a1 = carry
                a0 = a0 + rows_v[slot, rb + j, 0:HALF]
                a1 = a1 + rows_v[slot, rb + j, HALF:D]
                return a0, a1

            a0, a1 = pl.loop(0, L, init_carry=(bias_lo, bias_hi),
                             unroll=8)(body)
            acc_v[slot, r, 0:HALF] = a0
            acc_v[slot, r, HALF:D] = a1
        pltpu.async_copy(acc_v.at[slot],
                         out_hbm.at[pl.ds(base_row + ch * CB, CB), :],
                         osems[slot])

    load_and_fire(0, 0)

    @pl.loop(0, NCH, step=2)
    def _(c):
        for s in range(2):
            ch = c + s

            @pl.when(ch + 1 < NCH)
            def _():
                load_and_fire(ch + 1, (s + 1) % 2)

            pltpu.make_async_copy(
                embed_hbm.at[idx_v.at[pl.ds(s * ROWS, ROWS)]],
                rows_v.at[s], sems[s]).wait()
            accumulate(ch, s)

    for s in range(2):
        pltpu.make_async_copy(
            acc_v.at[s],
            out_hbm.at[pl.ds(base_row + (NCH - 2 + s) * CB, CB), :],
            osems[s]).wait()


_TC_CC = 16384
_TC_QQ = _TC_CC // 4


_TC_NCHUNK = 4
_TC_SC = _TC_CC // _TC_NCHUNK
_TC_SQ = _TC_SC // 4


def _pack_body(in_ref, out_ref, *ys):
    for c in range(_TC_NCHUNK):
        ys[c][...] = in_ref[:, c * _TC_SC:(c + 1) * _TC_SC].T
    for c in range(_TC_NCHUNK):
        for j in range(4):
            out_ref[c * _TC_SQ:(c + 1) * _TC_SQ, D * j:D * (j + 1)] = (
                ys[c][pl.Slice(j, _TC_SQ, 4), :])


def _pack_table(embed):
    return pl.pallas_call(
        _pack_body,
        grid=((VOCAB + _TC_CC - 1) // _TC_CC,),
        in_specs=[pl.BlockSpec((D, _TC_CC), lambda c: (0, c))],
        out_specs=pl.BlockSpec((_TC_QQ, 128), lambda c: (c, 0)),
        out_shape=jax.ShapeDtypeStruct((VOCAB // 4, 128), jnp.float32),
        scratch_shapes=[pltpu.VMEM((_TC_SC, D), jnp.float32)
                        for _ in range(_TC_NCHUNK)],
    )(embed.T)


@jax.jit
def kernel(inputs, embed, bias):
    idx2d = inputs.astype(jnp.int32)
    tbl = _pack_table(embed).reshape(VOCAB, D)
    mesh = plsc.VectorSubcoreMesh(core_axis_name="c", subcore_axis_name="s")
    run = pl.kernel(
        _bow_kernel,
        out_type=jax.ShapeDtypeStruct((B, D), jnp.float32),
        mesh=mesh,
        scratch_types=[
            pltpu.VMEM((2 * ROWS,), jnp.int32),
            pltpu.VMEM((2, ROWS, D), jnp.float32),
            pltpu.VMEM((2, CB, D), jnp.float32),
            pltpu.VMEM((D,), jnp.float32),
            pltpu.SemaphoreType.DMA,
            pltpu.SemaphoreType.DMA,
            pltpu.SemaphoreType.DMA,
            pltpu.SemaphoreType.DMA,
            pltpu.SemaphoreType.DMA,
            pltpu.SemaphoreType.DMA,
        ],
        compiler_params=pltpu.CompilerParams(use_tc_tiling_on_sc=False),
    )
    return run(idx2d, tbl, bias)


# speedup vs baseline: 23.2715x; 1.0037x over previous
import functools

import jax
import jax.numpy as jnp
from jax import lax
from jax.experimental import pallas as pl
from jax.experimental.pallas import tpu as pltpu
from jax.experimental.pallas import tpu_sc as plsc

B = 16384
L = 200
D = 32
VOCAB = 1000000

NC = 2
NS = 16
NW = NC * NS
BPW = B // NW
CB = 8
NCH = BPW // CB
ROWS = CB * L
HALF = D // 2


def _bow_kernel(idx_hbm, embed_hbm, bias_hbm, out_hbm,
                idx_v, rows_v, acc_v, bias_v, sem0, sem1, isem0, isem1,
                osem0, osem1):
    wid = lax.axis_index("s") * NC + lax.axis_index("c")
    base_row = wid * BPW
    sems = (sem0, sem1)
    isems = (isem0, isem1)
    osems = (osem0, osem1)

    pltpu.sync_copy(bias_hbm, bias_v)
    bias_lo = bias_v[0:HALF]
    bias_hi = bias_v[HALF:D]

    def load_and_fire(ch, slot):
        row0 = base_row + ch * CB
        for r in range(CB):
            pltpu.async_copy(idx_hbm.at[row0 + r, :],
                             idx_v.at[pl.ds(slot * ROWS + r * L, L)],
                             isems[slot])
        for r in range(CB):
            pltpu.make_async_copy(idx_hbm.at[row0 + r, :],
                                  idx_v.at[pl.ds(slot * ROWS + r * L, L)],
                                  isems[slot]).wait()
        return pltpu.async_copy(embed_hbm.at[idx_v.at[pl.ds(slot * ROWS, ROWS)]],
                                rows_v.at[slot], sems[slot])

    def accumulate(ch, slot):
        @pl.when(ch >= 2)
        def _():
            pltpu.make_async_copy(
                acc_v.at[slot],
                out_hbm.at[pl.ds(base_row + (ch - 2) * CB, CB), :],
                osems[slot]).wait()

        for r in range(CB):
            rb = r * L

            def body(j, carry):
                a0, a1 = carry
                a0 = a0 + rows_v[slot, rb + j, 0:HALF]
                a1 = a1 + rows_v[slot, rb + j, HALF:D]
                return a0, a1

            a0, a1 = pl.loop(0, L, init_carry=(bias_lo, bias_hi),
                             unroll=8)(body)
            acc_v[slot, r, 0:HALF] = a0
            acc_v[slot, r, HALF:D] = a1
        pltpu.async_copy(acc_v.at[slot],
                         out_hbm.at[pl.ds(base_row + ch * CB, CB), :],
                         osems[slot])

    load_and_fire(0, 0)

    @pl.loop(0, NCH, step=2)
    def _(c):
        for s in range(2):
            ch = c + s

            @pl.when(ch + 1 < NCH)
            def _():
                load_and_fire(ch + 1, (s + 1) % 2)

            pltpu.make_async_copy(
                embed_hbm.at[idx_v.at[pl.ds(s * ROWS, ROWS)]],
                rows_v.at[s], sems[s]).wait()
            accumulate(ch, s)

    for s in range(2):
        pltpu.make_async_copy(
            acc_v.at[s],
            out_hbm.at[pl.ds(base_row + (NCH - 2 + s) * CB, CB), :],
            osems[s]).wait()


_TC_CC = 32768
_TC_QQ = _TC_CC // 4


_TC_NCHUNK = 4
_TC_SC = _TC_CC // _TC_NCHUNK
_TC_SQ = _TC_SC // 4


def _pack_body(in_ref, out_ref, *ys):
    for c in range(_TC_NCHUNK):
        ys[c][...] = in_ref[:, c * _TC_SC:(c + 1) * _TC_SC].T
    for c in range(_TC_NCHUNK):
        for j in range(4):
            out_ref[c * _TC_SQ:(c + 1) * _TC_SQ, D * j:D * (j + 1)] = (
                ys[c][pl.Slice(j, _TC_SQ, 4), :])


def _pack_table(embed):
    return pl.pallas_call(
        _pack_body,
        grid=((VOCAB + _TC_CC - 1) // _TC_CC,),
        in_specs=[pl.BlockSpec((D, _TC_CC), lambda c: (0, c))],
        out_specs=pl.BlockSpec((_TC_QQ, 128), lambda c: (c, 0)),
        out_shape=jax.ShapeDtypeStruct((VOCAB // 4, 128), jnp.float32),
        scratch_shapes=[pltpu.VMEM((_TC_SC, D), jnp.float32)
                        for _ in range(_TC_NCHUNK)],
    )(embed.T)


@jax.jit
def kernel(inputs, embed, bias):
    idx2d = inputs.astype(jnp.int32)
    tbl = _pack_table(embed).reshape(VOCAB, D)
    mesh = plsc.VectorSubcoreMesh(core_axis_name="c", subcore_axis_name="s")
    run = pl.kernel(
        _bow_kernel,
        out_type=jax.ShapeDtypeStruct((B, D), jnp.float32),
        mesh=mesh,
        scratch_types=[
            pltpu.VMEM((2 * ROWS,), jnp.int32),
            pltpu.VMEM((2, ROWS, D), jnp.float32),
            pltpu.VMEM((2, CB, D), jnp.float32),
            pltpu.VMEM((D,), jnp.float32),
            pltpu.SemaphoreType.DMA,
            pltpu.SemaphoreType.DMA,
            pltpu.SemaphoreType.DMA,
            pltpu.SemaphoreType.DMA,
            pltpu.SemaphoreType.DMA,
            pltpu.SemaphoreType.DMA,
        ],
        compiler_params=pltpu.CompilerParams(use_tc_tiling_on_sc=False),
    )
    return run(idx2d, tbl, bias)


# speedup vs baseline: 23.2887x; 1.0007x over previous
import functools

import jax
import jax.numpy as jnp
from jax import lax
from jax.experimental import pallas as pl
from jax.experimental.pallas import tpu as pltpu
from jax.experimental.pallas import tpu_sc as plsc

B = 16384
L = 200
D = 32
VOCAB = 1000000

NC = 2
NS = 16
NW = NC * NS
BPW = B // NW
CB = 8
NCH = BPW // CB
ROWS = CB * L
HALF = D // 2


def _bow_kernel(idx_hbm, embed_hbm, bias_hbm, out_hbm,
                idx_v, rows_v, acc_v, bias_v, sem0, sem1, isem0, isem1,
                osem0, osem1):
    wid = lax.axis_index("s") * NC + lax.axis_index("c")
    base_row = wid * BPW
    sems = (sem0, sem1)
    isems = (isem0, isem1)
    osems = (osem0, osem1)

    pltpu.sync_copy(bias_hbm, bias_v)
    bias_lo = bias_v[0:HALF]
    bias_hi = bias_v[HALF:D]

    def load_and_fire(ch, slot):
        row0 = base_row + ch * CB
        for r in range(CB):
            pltpu.async_copy(idx_hbm.at[row0 + r, :],
                             idx_v.at[pl.ds(slot * ROWS + r * L, L)],
                             isems[slot])
        for r in range(CB):
            pltpu.make_async_copy(idx_hbm.at[row0 + r, :],
                                  idx_v.at[pl.ds(slot * ROWS + r * L, L)],
                                  isems[slot]).wait()
        return pltpu.async_copy(embed_hbm.at[idx_v.at[pl.ds(slot * ROWS, ROWS)]],
                                rows_v.at[slot], sems[slot])

    def accumulate(ch, slot):
        @pl.when(ch >= 2)
        def _():
            pltpu.make_async_copy(
                acc_v.at[slot],
                out_hbm.at[pl.ds(base_row + (ch - 2) * CB, CB), :],
                osems[slot]).wait()

        for r in range(CB):
            rb = r * L

            def body(j, carry):
                a0, a1 = carry
                a0 = a0 + rows_v[slot, rb + j, 0:HALF]
                a1 = a1 + rows_v[slot, rb + j, HALF:D]
                return a0, a1

            a0, a1 = pl.loop(0, L, init_carry=(bias_lo, bias_hi),
                             unroll=8)(body)
            acc_v[slot, r, 0:HALF] = a0
            acc_v[slot, r, HALF:D] = a1
        pltpu.async_copy(acc_v.at[slot],
                         out_hbm.at[pl.ds(base_row + ch * CB, CB), :],
                         osems[slot])

    load_and_fire(0, 0)

    @pl.loop(0, NCH, step=2)
    def _(c):
        for s in range(2):
            ch = c + s

            @pl.when(ch + 1 < NCH)
            def _():
                load_and_fire(ch + 1, (s + 1) % 2)

            pltpu.make_async_copy(
                embed_hbm.at[idx_v.at[pl.ds(s * ROWS, ROWS)]],
                rows_v.at[s], sems[s]).wait()
            accumulate(ch, s)

    for s in range(2):
        pltpu.make_async_copy(
            acc_v.at[s],
            out_hbm.at[pl.ds(base_row + (NCH - 2 + s) * CB, CB), :],
            osems[s]).wait()


_TC_CC = 32768
_TC_QQ = _TC_CC // 4


_TC_NCHUNK = 8
_TC_SC = _TC_CC // _TC_NCHUNK
_TC_SQ = _TC_SC // 4


def _pack_body(in_ref, out_ref, *ys):
    for c in range(_TC_NCHUNK):
        ys[c][...] = in_ref[:, c * _TC_SC:(c + 1) * _TC_SC].T
    for c in range(_TC_NCHUNK):
        for j in range(4):
            out_ref[c * _TC_SQ:(c + 1) * _TC_SQ, D * j:D * (j + 1)] = (
                ys[c][pl.Slice(j, _TC_SQ, 4), :])


def _pack_table(embed):
    return pl.pallas_call(
        _pack_body,
        grid=((VOCAB + _TC_CC - 1) // _TC_CC,),
        in_specs=[pl.BlockSpec((D, _TC_CC), lambda c: (0, c))],
        out_specs=pl.BlockSpec((_TC_QQ, 128), lambda c: (c, 0)),
        out_shape=jax.ShapeDtypeStruct((VOCAB // 4, 128), jnp.float32),
        scratch_shapes=[pltpu.VMEM((_TC_SC, D), jnp.float32)
                        for _ in range(_TC_NCHUNK)],
    )(embed.T)


@jax.jit
def kernel(inputs, embed, bias):
    idx2d = inputs.astype(jnp.int32)
    tbl = _pack_table(embed).reshape(VOCAB, D)
    mesh = plsc.VectorSubcoreMesh(core_axis_name="c", subcore_axis_name="s")
    run = pl.kernel(
        _bow_kernel,
        out_type=jax.ShapeDtypeStruct((B, D), jnp.float32),
        mesh=mesh,
        scratch_types=[
            pltpu.VMEM((2 * ROWS,), jnp.int32),
            pltpu.VMEM((2, ROWS, D), jnp.float32),
            pltpu.VMEM((2, CB, D), jnp.float32),
            pltpu.VMEM((D,), jnp.float32),
            pltpu.SemaphoreType.DMA,
            pltpu.SemaphoreType.DMA,
            pltpu.SemaphoreType.DMA,
            pltpu.SemaphoreType.DMA,
            pltpu.SemaphoreType.DMA,
            pltpu.SemaphoreType.DMA,
        ],
        compiler_params=pltpu.CompilerParams(use_tc_tiling_on_sc=False),
    )
    return run(idx2d, tbl, bias)
